# baseline (speedup 1.0000x reference)
import functools

import jax
import jax.numpy as jnp
from jax import lax
from jax.experimental import pallas as pl
from jax.experimental.pallas import tpu as pltpu
from jax.experimental.pallas import tpu_sc as plsc

_V = 1000000
_H = 64
_HP = 128
_B = 4096
_L = 200
_NC = 2
_NS = 16
_NW = _NC * _NS
_BAGS_W = _B // _NW
_CHUNK = 40
_NCHUNK = _L // _CHUNK
_NCH = _BAGS_W * _NCHUNK
_NBUF = 8

_mesh = plsc.VectorSubcoreMesh(core_axis_name="c", subcore_axis_name="s")


@functools.partial(
    pl.kernel,
    mesh=_mesh,
    out_type=jax.ShapeDtypeStruct((_B, _H), jnp.float32),
    scratch_types=[
        pltpu.VMEM((_NBUF, _CHUNK), jnp.int32),
        pltpu.VMEM((_NBUF, _CHUNK, _HP), jnp.float32),
        pltpu.VMEM((_BAGS_W, _H), jnp.float32),
    ] + [pltpu.SemaphoreType.DMA] * (2 * _NBUF),
    compiler_params=pltpu.CompilerParams(use_tc_tiling_on_sc=True),
)
def _sc_bag_sum(ids_hbm, table_hbm, out_hbm, idx_v, rows_v, sums_v, *sems):
    wid = lax.axis_index("s") * _NC + lax.axis_index("c")
    base = wid * _NCH
    gsem, isem = sems[:_NBUF], sems[_NBUF:]

    def start_idx(t, k):
        pltpu.async_copy(ids_hbm.at[base + t], idx_v.at[k], isem[k])

    def wait_idx(t, k):
        pltpu.make_async_copy(ids_hbm.at[base + t], idx_v.at[k],
                              isem[k]).wait()

    def start(t, k):
        wait_idx(t, k)
        pltpu.async_copy(table_hbm.at[idx_v.at[k]], rows_v.at[k], gsem[k])

    def wait(t, k):
        pltpu.make_async_copy(table_hbm.at[idx_v.at[k]], rows_v.at[k],
                              gsem[k]).wait()

    for k in range(_NBUF):
        start_idx(k, k)
    for k in range(_NBUF):
        start(k, k)

    def outer(i, accs):
        tt = i * _NBUF
        for k in range(_NBUF):
            t = tt + k
            j = lax.rem(t, _NCHUNK)
            bag = lax.div(t, _NCHUNK)
            wait(t, k)
            nxt = t + _NBUF

            @pl.when(nxt < _NCH)
            def _():
                start_idx(nxt, k)

            keep = (j != 0).astype(jnp.float32)
            accs = tuple(a * keep for a in accs)

            def red(m, a):
                for u in range(8):
                    r = m * 8 + u
                    a = tuple(a[q] + rows_v[k, r, pl.ds(q * 16, 16)]
                              for q in range(4))
                return a

            accs = lax.fori_loop(0, _CHUNK // 8, red, accs)

            @pl.when(nxt < _NCH)
            def _():
                start(nxt, k)

            @pl.when(j == _NCHUNK - 1)
            def _():
                for q in range(4):
                    sums_v[bag, pl.ds(q * 16, 16)] = accs[q]
        return accs

    lax.fori_loop(0, _NCH // _NBUF, outer, (jnp.zeros((16,), jnp.float32),) * 4)
    pltpu.sync_copy(sums_v, out_hbm.at[pl.ds(wid * _BAGS_W, _BAGS_W)])


_VB = 32768


def _tr_body(t_ref, o_ref):
    o_ref[:, pl.ds(0, _H)] = t_ref[...].T


def _relayout_table(tT):
    return pl.pallas_call(
        _tr_body,
        grid=(pl.cdiv(_V, _VB),),
        in_specs=[pl.BlockSpec((_H, _VB), lambda i: (0, i))],
        out_specs=pl.BlockSpec((_VB, _HP), lambda i: (i, 0)),
        out_shape=jax.ShapeDtypeStruct((_V, _HP), jnp.float32),
    )(tT)


def _tc_body(x_ref, w_ref, b_ref, o_ref):
    x = jnp.maximum(x_ref[...] * (1.0 / _L), 0.0)
    o_ref[...] = lax.dot_general(
        x, w_ref[...], (((1,), (1,)), ((), ())),
        preferred_element_type=jnp.float32) + b_ref[...]


def kernel(input_ids, emb_table, W, b):
    ids = input_ids.astype(jnp.int32).reshape(_B * _NCHUNK, _CHUNK)
    table = _relayout_table(emb_table.T)
    sums = _sc_bag_sum(ids, table)
    h = pl.pallas_call(
        _tc_body,
        out_shape=jax.ShapeDtypeStruct((_B, _H), jnp.float32),
    )(sums, W, b.reshape(1, _H))
    return (h[:, None, :],)


# speedup vs baseline: 1.2683x; 1.2683x over previous
import functools

import jax
import jax.numpy as jnp
from jax import lax
from jax.experimental import pallas as pl
from jax.experimental.pallas import tpu as pltpu
from jax.experimental.pallas import tpu_sc as plsc

_V = 1000000
_H = 64
_HP = 128
_B = 4096
_L = 200
_NC = 2
_NS = 16
_NW = _NC * _NS
_BAGS_W = _B // _NW
_CHUNK = 40
_NCHUNK = _L // _CHUNK
_NCH = _BAGS_W * _NCHUNK
_NBUF = 4

_mesh = plsc.VectorSubcoreMesh(core_axis_name="c", subcore_axis_name="s")


@functools.partial(
    pl.kernel,
    mesh=_mesh,
    out_type=jax.ShapeDtypeStruct((_B, _H), jnp.float32),
    scratch_types=[
        pltpu.VMEM((_NCH, _CHUNK), jnp.int32),
        pltpu.VMEM((_NBUF, _CHUNK, _HP), jnp.float32),
        pltpu.VMEM((_BAGS_W, _H), jnp.float32),
    ] + [pltpu.SemaphoreType.DMA] * _NBUF,
    compiler_params=pltpu.CompilerParams(use_tc_tiling_on_sc=True),
)
def _sc_bag_sum(ids_hbm, table_hbm, out_hbm, idx_v, rows_v, sums_v, *sems):
    wid = lax.axis_index("s") * _NC + lax.axis_index("c")
    pltpu.sync_copy(ids_hbm.at[pl.ds(wid * _NCH, _NCH)], idx_v)

    def start(t, k):
        pltpu.async_copy(table_hbm.at[idx_v.at[t]], rows_v.at[k], sems[k])

    def wait(t, k):
        pltpu.make_async_copy(table_hbm.at[idx_v.at[t]], rows_v.at[k],
                              sems[k]).wait()

    for k in range(_NBUF):
        start(k, k)

    def outer(i, accs):
        tt = i * _NBUF
        for k in range(_NBUF):
            t = tt + k
            j = lax.rem(t, _NCHUNK)
            bag = lax.div(t, _NCHUNK)
            wait(t, k)
            keep = (j != 0).astype(jnp.float32)
            accs = tuple(a * keep for a in accs)
            for r in range(_CHUNK):
                accs = tuple(accs[q] + rows_v[k, r, pl.ds(q * 16, 16)]
                             for q in range(4))
            nxt = t + _NBUF

            @pl.when(nxt < _NCH)
            def _():
                start(nxt, k)

            @pl.when(j == _NCHUNK - 1)
            def _():
                for q in range(4):
                    sums_v[bag, pl.ds(q * 16, 16)] = accs[q]
        return accs

    lax.fori_loop(0, _NCH // _NBUF, outer, (jnp.zeros((16,), jnp.float32),) * 4)
    pltpu.sync_copy(sums_v, out_hbm.at[pl.ds(wid * _BAGS_W, _BAGS_W)])


_VB = 32768


def _tr_body(t_ref, o_ref):
    o_ref[:, pl.ds(0, _H)] = t_ref[...].T


def _relayout_table(tT):
    return pl.pallas_call(
        _tr_body,
        grid=(pl.cdiv(_V, _VB),),
        in_specs=[pl.BlockSpec((_H, _VB), lambda i: (0, i))],
        out_specs=pl.BlockSpec((_VB, _HP), lambda i: (i, 0)),
        out_shape=jax.ShapeDtypeStruct((_V, _HP), jnp.float32),
    )(tT)


def _tc_body(x_ref, w_ref, b_ref, o_ref):
    x = jnp.maximum(x_ref[...] * (1.0 / _L), 0.0)
    o_ref[...] = lax.dot_general(
        x, w_ref[...], (((1,), (1,)), ((), ())),
        preferred_element_type=jnp.float32) + b_ref[...]


def kernel(input_ids, emb_table, W, b):
    ids = input_ids.astype(jnp.int32).reshape(_B * _NCHUNK, _CHUNK)
    table = _relayout_table(emb_table.T)
    sums = _sc_bag_sum(ids, table)
    h = pl.pallas_call(
        _tc_body,
        out_shape=jax.ShapeDtypeStruct((_B, _H), jnp.float32),
    )(sums, W, b.reshape(1, _H))
    return (h[:, None, :],)
